# Initial kernel scaffold; baseline (speedup 1.0000x reference)
import jax
import jax.numpy as jnp
from jax.experimental import pallas as pl


def kernel(x, edge_index, W1, att_src1, att_dst1, bias1, W2, att_src2, att_dst2, bias2):
    raise NotImplementedError("write your pallas kernel here")


# speedup vs baseline: 13.6857x; 13.6857x over previous
import functools

import jax
import jax.numpy as jnp
from jax import lax
from jax.experimental import pallas as pl
from jax.experimental.pallas import tpu as pltpu
from jax.experimental.pallas import tpu_sc as plsc

N = 10000
E = 320000
C = 128
NC = 2
NS = 16
CH = C // NC
EW = E // NS
K = 80
NB = EW // K


def _prep_body(x_ref, w_ref, as_ref, ad_ref, xh_ref, asrc_ref, adst_ref):
    xh = lax.dot_general(x_ref[...], w_ref[...],
                         (((1,), (1,)), ((), ())),
                         preferred_element_type=jnp.float32)
    xh_ref[0] = xh[:, :CH]
    xh_ref[1] = xh[:, CH:]
    asrc_ref[...] = lax.dot_general(xh, as_ref[...],
                                    (((1,), (1,)), ((), ())),
                                    preferred_element_type=jnp.float32)[:, 0]
    adst_ref[...] = lax.dot_general(xh, ad_ref[...],
                                    (((1,), (1,)), ((), ())),
                                    preferred_element_type=jnp.float32)[:, 0]


def _tc_prep(x, w, att_s, att_d):
    return pl.pallas_call(
        _prep_body,
        out_shape=[
            jax.ShapeDtypeStruct((NC, N, CH), jnp.float32),
            jax.ShapeDtypeStruct((N,), jnp.float32),
            jax.ShapeDtypeStruct((N,), jnp.float32),
        ],
    )(x, w, att_s.reshape(1, C), att_d.reshape(1, C))


def _mid_body(acc_ref, den_ref, b_ref, w_ref, as_ref, ad_ref,
              xh_ref, asrc_ref, adst_ref):
    den = den_ref[...] + 1e-16
    num = jnp.concatenate((acc_ref[0], acc_ref[1]), axis=-1)
    h = num / den[:, None] + b_ref[...][None, :]
    h = jnp.maximum(h, 0.0)
    xh = lax.dot_general(h, w_ref[...], (((1,), (1,)), ((), ())),
                         preferred_element_type=jnp.float32)
    xh_ref[0] = xh[:, :CH]
    xh_ref[1] = xh[:, CH:]
    asrc_ref[...] = lax.dot_general(xh, as_ref[...],
                                    (((1,), (1,)), ((), ())),
                                    preferred_element_type=jnp.float32)[:, 0]
    adst_ref[...] = lax.dot_general(xh, ad_ref[...],
                                    (((1,), (1,)), ((), ())),
                                    preferred_element_type=jnp.float32)[:, 0]


def _tc_mid(acc, den, bias, w, att_s, att_d):
    return pl.pallas_call(
        _mid_body,
        out_shape=[
            jax.ShapeDtypeStruct((NC, N, CH), jnp.float32),
            jax.ShapeDtypeStruct((N,), jnp.float32),
            jax.ShapeDtypeStruct((N,), jnp.float32),
        ],
    )(acc, den, bias, w, att_s.reshape(1, C), att_d.reshape(1, C))


def _final_body(acc_ref, den_ref, b_ref, out_ref):
    den = den_ref[...] + 1e-16
    num = jnp.concatenate((acc_ref[0], acc_ref[1]), axis=-1)
    out_ref[...] = num / den[:, None] + b_ref[...][None, :]


def _tc_final(acc, den, bias):
    return pl.pallas_call(
        _final_body,
        out_shape=jax.ShapeDtypeStruct((N, C), jnp.float32),
    )(acc, den, bias)


def _sc_edge_body(xh_hbm, src_hbm, dst_hbm, asrc_hbm, adst_hbm,
                  acc_hbm, den_hbm,
                  src_v, dst_v, a1_v, a2_v, w_v, rows_v, zb_v, dz_v,
                  acc_sp, den_sp, sem):
    c = lax.axis_index("c")
    s = lax.axis_index("s")

    def _z(j, _):
        for r in range(CH // 16):
            zb_v[j, pl.ds(16 * r, 16)] = jnp.zeros((16,), jnp.float32)
        return 0
    lax.fori_loop(0, 200, _z, 0)

    def _zd(j, _):
        dz_v[pl.ds(16 * j, 16)] = jnp.zeros((16,), jnp.float32)
        return 0
    lax.fori_loop(0, 125, _zd, 0)

    @pl.when(s < 10)
    def _zero_acc():
        for i in range(5):
            pltpu.sync_copy(zb_v, acc_sp.at[pl.ds(s * 1000 + i * 200, 200)])

    @pl.when(s == 0)
    def _zero_den():
        for i in range(N // 2000):
            pltpu.sync_copy(dz_v, den_sp.at[pl.ds(i * 2000, 2000)])

    pltpu.sync_copy(src_hbm.at[s], src_v)
    pltpu.sync_copy(dst_hbm.at[s], dst_v)

    plsc.subcore_barrier()

    def _block(b, _):
        cp_rows = pltpu.async_copy(xh_hbm.at[c].at[src_v.at[b]], rows_v, sem)
        cp_a1 = pltpu.async_copy(asrc_hbm.at[src_v.at[b]], a1_v, sem)
        cp_a2 = pltpu.async_copy(adst_hbm.at[dst_v.at[b]], a2_v, sem)
        cp_rows.wait()
        cp_a1.wait()
        cp_a2.wait()

        for g in range(K // 16):
            sl = pl.ds(16 * g, 16)
            v = a1_v[sl] + a2_v[sl]
            w_v[sl] = jnp.exp(jnp.maximum(v, 0.2 * v))

        def _scale(g, _):
            w16 = w_v[pl.ds(16 * g, 16)]
            for l in range(16):
                j = 16 * g + l
                wj = w16[l]
                for r in range(CH // 16):
                    sl = pl.ds(16 * r, 16)
                    rows_v[j, sl] = rows_v[j, sl] * wj
            return 0
        lax.fori_loop(0, K // 16, _scale, 0)

        pltpu.sync_copy(rows_v, acc_sp.at[dst_v.at[b]], add=True)
        pltpu.sync_copy(w_v, den_sp.at[dst_v.at[b]], add=True)
        return 0

    lax.fori_loop(0, NB, _block, 0)

    plsc.subcore_barrier()

    @pl.when(s < 10)
    def _export_acc():
        pltpu.sync_copy(acc_sp.at[pl.ds(s * 1000, 1000)],
                        acc_hbm.at[c, pl.ds(s * 1000, 1000)])

    @pl.when(jnp.logical_and(s == 0, c == 0))
    def _export_den():
        pltpu.sync_copy(den_sp, den_hbm)


def _sc_edge(xh, src, dst, asrc, adst):
    f = pl.kernel(
        _sc_edge_body,
        out_type=[
            jax.ShapeDtypeStruct((NC, N, CH), jnp.float32),
            jax.ShapeDtypeStruct((N,), jnp.float32),
        ],
        mesh=plsc.VectorSubcoreMesh(core_axis_name="c", subcore_axis_name="s"),
        compiler_params=pltpu.CompilerParams(use_tc_tiling_on_sc=False),
        scratch_types=[
            pltpu.VMEM((NB, K), jnp.int32),
            pltpu.VMEM((NB, K), jnp.int32),
            pltpu.VMEM((K,), jnp.float32),
            pltpu.VMEM((K,), jnp.float32),
            pltpu.VMEM((K,), jnp.float32),
            pltpu.VMEM((K, CH), jnp.float32),
            pltpu.VMEM((200, CH), jnp.float32),
            pltpu.VMEM((2000,), jnp.float32),
            pltpu.VMEM_SHARED((N, CH), jnp.float32),
            pltpu.VMEM_SHARED((N,), jnp.float32),
            pltpu.SemaphoreType.DMA,
        ],
    )
    return f(xh, src, dst, asrc, adst)


def kernel(x, edge_index, W1, att_src1, att_dst1, bias1,
           W2, att_src2, att_dst2, bias2):
    ei = edge_index.astype(jnp.int32).reshape(2, NS, NB, K)
    src, dst = ei[0], ei[1]

    xh1, asrc1, adst1 = _tc_prep(x, W1, att_src1, att_dst1)
    acc1, den1 = _sc_edge(xh1, src, dst, asrc1, adst1)
    xh2, asrc2, adst2 = _tc_mid(acc1, den1, bias1, W2, att_src2, att_dst2)
    acc2, den2 = _sc_edge(xh2, src, dst, asrc2, adst2)
    return _tc_final(acc2, den2, bias2)


# speedup vs baseline: 17.4943x; 1.2783x over previous
import functools

import jax
import jax.numpy as jnp
from jax import lax
from jax.experimental import pallas as pl
from jax.experimental.pallas import tpu as pltpu
from jax.experimental.pallas import tpu_sc as plsc

N = 10000
E = 320000
C = 128
NC = 2
NS = 16
CH = C // NC
EW = E // NS
K = 80
NB = EW // K


def _prep_body(x_ref, w_ref, as_ref, ad_ref, xh_ref, asrc_ref, adst_ref):
    xh = lax.dot_general(x_ref[...], w_ref[...],
                         (((1,), (1,)), ((), ())),
                         preferred_element_type=jnp.float32)
    xh_ref[0] = xh[:, :CH]
    xh_ref[1] = xh[:, CH:]
    asrc_ref[...] = lax.dot_general(xh, as_ref[...],
                                    (((1,), (1,)), ((), ())),
                                    preferred_element_type=jnp.float32)[:, 0]
    adst_ref[...] = lax.dot_general(xh, ad_ref[...],
                                    (((1,), (1,)), ((), ())),
                                    preferred_element_type=jnp.float32)[:, 0]


def _tc_prep(x, w, att_s, att_d):
    return pl.pallas_call(
        _prep_body,
        out_shape=[
            jax.ShapeDtypeStruct((NC, N, CH), jnp.float32),
            jax.ShapeDtypeStruct((N,), jnp.float32),
            jax.ShapeDtypeStruct((N,), jnp.float32),
        ],
    )(x, w, att_s.reshape(1, C), att_d.reshape(1, C))


def _mid_body(acc_ref, den_ref, b_ref, w_ref, as_ref, ad_ref,
              xh_ref, asrc_ref, adst_ref):
    den = den_ref[...] + 1e-16
    num = jnp.concatenate((acc_ref[0], acc_ref[1]), axis=-1)
    h = num / den[:, None] + b_ref[...][None, :]
    h = jnp.maximum(h, 0.0)
    xh = lax.dot_general(h, w_ref[...], (((1,), (1,)), ((), ())),
                         preferred_element_type=jnp.float32)
    xh_ref[0] = xh[:, :CH]
    xh_ref[1] = xh[:, CH:]
    asrc_ref[...] = lax.dot_general(xh, as_ref[...],
                                    (((1,), (1,)), ((), ())),
                                    preferred_element_type=jnp.float32)[:, 0]
    adst_ref[...] = lax.dot_general(xh, ad_ref[...],
                                    (((1,), (1,)), ((), ())),
                                    preferred_element_type=jnp.float32)[:, 0]


def _tc_mid(acc, den, bias, w, att_s, att_d):
    return pl.pallas_call(
        _mid_body,
        out_shape=[
            jax.ShapeDtypeStruct((NC, N, CH), jnp.float32),
            jax.ShapeDtypeStruct((N,), jnp.float32),
            jax.ShapeDtypeStruct((N,), jnp.float32),
        ],
    )(acc, den, bias, w, att_s.reshape(1, C), att_d.reshape(1, C))


def _final_body(acc_ref, den_ref, b_ref, out_ref):
    den = den_ref[...] + 1e-16
    num = jnp.concatenate((acc_ref[0], acc_ref[1]), axis=-1)
    out_ref[...] = num / den[:, None] + b_ref[...][None, :]


def _tc_final(acc, den, bias):
    return pl.pallas_call(
        _final_body,
        out_shape=jax.ShapeDtypeStruct((N, C), jnp.float32),
    )(acc, den, bias)


def _sc_edge_body(xh_hbm, src_hbm, dst_hbm, asrc_hbm, adst_hbm,
                  acc_hbm, den_hbm,
                  src_v, dst_v, a1b, a2b, w_all, rows0, rows1,
                  zb_v, dz_v, acc_sp, den_sp,
                  gsem0, gsem1, ssem0, ssem1, dsem):
    c = lax.axis_index("c")
    s = lax.axis_index("s")

    def _z(j, _):
        for r in range(CH // 16):
            zb_v[j, pl.ds(16 * r, 16)] = jnp.zeros((16,), jnp.float32)
        return 0
    lax.fori_loop(0, 200, _z, 0)

    def _zd(j, _):
        dz_v[pl.ds(16 * j, 16)] = jnp.zeros((16,), jnp.float32)
        return 0
    lax.fori_loop(0, 125, _zd, 0)

    @pl.when(s < 10)
    def _zero_acc():
        for i in range(5):
            pltpu.sync_copy(zb_v, acc_sp.at[pl.ds(s * 1000 + i * 200, 200)])

    @pl.when(s == 0)
    def _zero_den():
        for i in range(N // 2000):
            pltpu.sync_copy(dz_v, den_sp.at[pl.ds(i * 2000, 2000)])

    pltpu.sync_copy(src_hbm.at[s], src_v)
    pltpu.sync_copy(dst_hbm.at[s], dst_v)

    plsc.subcore_barrier()

    def _p1(t, _):
        cps = []
        for i in range(25):
            b = 25 * t + i
            cps.append(pltpu.async_copy(
                asrc_hbm.at[src_v.at[b]], a1b.at[pl.ds(K * i, K)], dsem))
            cps.append(pltpu.async_copy(
                adst_hbm.at[dst_v.at[b]], a2b.at[pl.ds(K * i, K)], dsem))
        for cp in cps:
            cp.wait()

        def _w(j, _):
            v = a1b[pl.ds(16 * j, 16)] + a2b[pl.ds(16 * j, 16)]
            w_all[pl.ds(2000 * t + 16 * j, 16)] = (
                jnp.exp(jnp.maximum(v, 0.2 * v)))
            return 0
        lax.fori_loop(0, 125, _w, 0)

        cps = []
        for i in range(25):
            b = 25 * t + i
            cps.append(pltpu.async_copy(
                w_all.at[pl.ds(K * b, K)], den_sp.at[dst_v.at[b]],
                dsem, add=True))
        for cp in cps:
            cp.wait()
        return 0
    lax.fori_loop(0, NB // 25, _p1, 0)

    def _scale_rows(rows_v, b):
        def _scale(g, _):
            w16 = w_all[pl.ds(K * b + 16 * g, 16)]
            for l in range(16):
                j = 16 * g + l
                wj = w16[l]
                for r in range(CH // 16):
                    sl = pl.ds(16 * r, 16)
                    rows_v[j, sl] = rows_v[j, sl] * wj
            return 0
        lax.fori_loop(0, K // 16, _scale, 0)

    def _start_gather(b, buf, sem):
        pltpu.async_copy(xh_hbm.at[c].at[src_v.at[b]], buf, sem)

    def _wait_gather(buf, sem):
        pltpu.make_async_copy(xh_hbm.at[c].at[src_v.at[0]], buf, sem).wait()

    def _start_scatter(b, buf, sem):
        pltpu.async_copy(buf, acc_sp.at[dst_v.at[b]], sem, add=True)

    def _wait_scatter(buf, sem):
        pltpu.make_async_copy(buf, acc_sp.at[dst_v.at[0]], sem).wait()

    _start_gather(0, rows0, gsem0)

    def _p2(t, _):
        b0 = 2 * t
        @pl.when(t > 0)
        def _drain1():
            _wait_scatter(rows1, ssem1)
        _start_gather(b0 + 1, rows1, gsem1)
        _wait_gather(rows0, gsem0)
        _scale_rows(rows0, b0)
        _start_scatter(b0, rows0, ssem0)
        _wait_scatter(rows0, ssem0)
        @pl.when(t < NB // 2 - 1)
        def _next():
            _start_gather(b0 + 2, rows0, gsem0)
        _wait_gather(rows1, gsem1)
        _scale_rows(rows1, b0 + 1)
        _start_scatter(b0 + 1, rows1, ssem1)
        return 0

    lax.fori_loop(0, NB // 2, _p2, 0)
    _wait_scatter(rows1, ssem1)

    plsc.subcore_barrier()

    @pl.when(s < 10)
    def _export_acc():
        pltpu.sync_copy(acc_sp.at[pl.ds(s * 1000, 1000)],
                        acc_hbm.at[c, pl.ds(s * 1000, 1000)])

    @pl.when(jnp.logical_and(s == 0, c == 0))
    def _export_den():
        pltpu.sync_copy(den_sp, den_hbm)


def _sc_edge(xh, src, dst, asrc, adst):
    f = pl.kernel(
        _sc_edge_body,
        out_type=[
            jax.ShapeDtypeStruct((NC, N, CH), jnp.float32),
            jax.ShapeDtypeStruct((N,), jnp.float32),
        ],
        mesh=plsc.VectorSubcoreMesh(core_axis_name="c", subcore_axis_name="s"),
        compiler_params=pltpu.CompilerParams(use_tc_tiling_on_sc=False),
        scratch_types=[
            pltpu.VMEM((NB, K), jnp.int32),
            pltpu.VMEM((NB, K), jnp.int32),
            pltpu.VMEM((2000,), jnp.float32),
            pltpu.VMEM((2000,), jnp.float32),
            pltpu.VMEM((EW,), jnp.float32),
            pltpu.VMEM((K, CH), jnp.float32),
            pltpu.VMEM((K, CH), jnp.float32),
            pltpu.VMEM((200, CH), jnp.float32),
            pltpu.VMEM((2000,), jnp.float32),
            pltpu.VMEM_SHARED((N, CH), jnp.float32),
            pltpu.VMEM_SHARED((N,), jnp.float32),
            pltpu.SemaphoreType.DMA,
            pltpu.SemaphoreType.DMA,
            pltpu.SemaphoreType.DMA,
            pltpu.SemaphoreType.DMA,
            pltpu.SemaphoreType.DMA,
        ],
    )
    return f(xh, src, dst, asrc, adst)


def kernel(x, edge_index, W1, att_src1, att_dst1, bias1,
           W2, att_src2, att_dst2, bias2):
    ei = edge_index.astype(jnp.int32).reshape(2, NS, NB, K)
    src, dst = ei[0], ei[1]

    xh1, asrc1, adst1 = _tc_prep(x, W1, att_src1, att_dst1)
    acc1, den1 = _sc_edge(xh1, src, dst, asrc1, adst1)
    xh2, asrc2, adst2 = _tc_mid(acc1, den1, bias1, W2, att_src2, att_dst2)
    acc2, den2 = _sc_edge(xh2, src, dst, asrc2, adst2)
    return _tc_final(acc2, den2, bias2)


# speedup vs baseline: 17.5028x; 1.0005x over previous
import functools

import jax
import jax.numpy as jnp
from jax import lax
from jax.experimental import pallas as pl
from jax.experimental.pallas import tpu as pltpu
from jax.experimental.pallas import tpu_sc as plsc

N = 10000
E = 320000
C = 128
NC = 2
NS = 16
CH = C // NC
EW = E // NS
K = 80
NB = EW // K


def _prep_body(x_ref, w_ref, as_ref, ad_ref, xh_ref, asrc_ref, adst_ref):
    xh = lax.dot_general(x_ref[...], w_ref[...],
                         (((1,), (1,)), ((), ())),
                         preferred_element_type=jnp.float32)
    xh_ref[0] = xh[:, :CH]
    xh_ref[1] = xh[:, CH:]
    asrc_ref[...] = lax.dot_general(xh, as_ref[...],
                                    (((1,), (1,)), ((), ())),
                                    preferred_element_type=jnp.float32)[:, 0]
    adst_ref[...] = lax.dot_general(xh, ad_ref[...],
                                    (((1,), (1,)), ((), ())),
                                    preferred_element_type=jnp.float32)[:, 0]


def _tc_prep(x, w, att_s, att_d):
    return pl.pallas_call(
        _prep_body,
        out_shape=[
            jax.ShapeDtypeStruct((NC, N, CH), jnp.float32),
            jax.ShapeDtypeStruct((N,), jnp.float32),
            jax.ShapeDtypeStruct((N,), jnp.float32),
        ],
    )(x, w, att_s.reshape(1, C), att_d.reshape(1, C))


def _mid_body(acc_ref, den_ref, b_ref, w_ref, as_ref, ad_ref,
              xh_ref, asrc_ref, adst_ref):
    den = den_ref[...] + 1e-16
    num = jnp.concatenate((acc_ref[0], acc_ref[1]), axis=-1)
    h = num / den[:, None] + b_ref[...][None, :]
    h = jnp.maximum(h, 0.0)
    xh = lax.dot_general(h, w_ref[...], (((1,), (1,)), ((), ())),
                         preferred_element_type=jnp.float32)
    xh_ref[0] = xh[:, :CH]
    xh_ref[1] = xh[:, CH:]
    asrc_ref[...] = lax.dot_general(xh, as_ref[...],
                                    (((1,), (1,)), ((), ())),
                                    preferred_element_type=jnp.float32)[:, 0]
    adst_ref[...] = lax.dot_general(xh, ad_ref[...],
                                    (((1,), (1,)), ((), ())),
                                    preferred_element_type=jnp.float32)[:, 0]


def _tc_mid(acc, den, bias, w, att_s, att_d):
    return pl.pallas_call(
        _mid_body,
        out_shape=[
            jax.ShapeDtypeStruct((NC, N, CH), jnp.float32),
            jax.ShapeDtypeStruct((N,), jnp.float32),
            jax.ShapeDtypeStruct((N,), jnp.float32),
        ],
    )(acc, den, bias, w, att_s.reshape(1, C), att_d.reshape(1, C))


def _final_body(acc_ref, den_ref, b_ref, out_ref):
    den = den_ref[...] + 1e-16
    num = jnp.concatenate((acc_ref[0], acc_ref[1]), axis=-1)
    out_ref[...] = num / den[:, None] + b_ref[...][None, :]


def _tc_final(acc, den, bias):
    return pl.pallas_call(
        _final_body,
        out_shape=jax.ShapeDtypeStruct((N, C), jnp.float32),
    )(acc, den, bias)


def _sc_edge_body(xh_hbm, src_hbm, dst_hbm, asrc_hbm, adst_hbm,
                  acc_hbm, den_hbm,
                  src_v, dst_v, a1b, a2b, w_all, rows0, rows1,
                  zb_v, dz_v, acc_sp, den_sp,
                  gsem0, gsem1, ssem0, ssem1, dsem):
    c = lax.axis_index("c")
    s = lax.axis_index("s")

    def _z(j, _):
        for r in range(CH // 16):
            zb_v[j, pl.ds(16 * r, 16)] = jnp.zeros((16,), jnp.float32)
        return 0
    lax.fori_loop(0, 200, _z, 0)

    def _zd(j, _):
        dz_v[pl.ds(16 * j, 16)] = jnp.zeros((16,), jnp.float32)
        return 0
    lax.fori_loop(0, 125, _zd, 0)

    @pl.when(s < 10)
    def _zero_acc():
        for i in range(5):
            pltpu.sync_copy(zb_v, acc_sp.at[pl.ds(s * 1000 + i * 200, 200)])

    @pl.when(s == 0)
    def _zero_den():
        for i in range(N // 2000):
            pltpu.sync_copy(dz_v, den_sp.at[pl.ds(i * 2000, 2000)])

    pltpu.sync_copy(src_hbm.at[s], src_v)
    pltpu.sync_copy(dst_hbm.at[s], dst_v)

    plsc.subcore_barrier()

    def _p1(t, _):
        cps = []
        for i in range(25):
            b = 25 * t + i
            cps.append(pltpu.async_copy(
                asrc_hbm.at[src_v.at[b]], a1b.at[pl.ds(K * i, K)], dsem))
            cps.append(pltpu.async_copy(
                adst_hbm.at[dst_v.at[b]], a2b.at[pl.ds(K * i, K)], dsem))
        for cp in cps:
            cp.wait()

        def _w(j, _):
            v = a1b[pl.ds(16 * j, 16)] + a2b[pl.ds(16 * j, 16)]
            w_all[pl.ds(2000 * t + 16 * j, 16)] = (
                jnp.exp(jnp.maximum(v, 0.2 * v)))
            return 0
        lax.fori_loop(0, 125, _w, 0)

        cps = []
        for i in range(25):
            b = 25 * t + i
            cps.append(pltpu.async_copy(
                w_all.at[pl.ds(K * b, K)], den_sp.at[dst_v.at[b]],
                dsem, add=True))
        for cp in cps:
            cp.wait()
        return 0
    lax.fori_loop(0, NB // 25, _p1, 0)

    def _scale_rows(rows_v, b):
        def _scale(g, _):
            w16 = w_all[pl.ds(K * b + 16 * g, 16)]
            for l in range(16):
                j = 16 * g + l
                wj = jnp.take(w16, jnp.full((16,), l, jnp.int32))
                for r in range(CH // 16):
                    sl = pl.ds(16 * r, 16)
                    rows_v[j, sl] = rows_v[j, sl] * wj
            return 0
        lax.fori_loop(0, K // 16, _scale, 0)

    def _start_gather(b, buf, sem):
        pltpu.async_copy(xh_hbm.at[c].at[src_v.at[b]], buf, sem)

    def _wait_gather(buf, sem):
        pltpu.make_async_copy(xh_hbm.at[c].at[src_v.at[0]], buf, sem).wait()

    def _start_scatter(b, buf, sem):
        pltpu.async_copy(buf, acc_sp.at[dst_v.at[b]], sem, add=True)

    def _wait_scatter(buf, sem):
        pltpu.make_async_copy(buf, acc_sp.at[dst_v.at[0]], sem).wait()

    _start_gather(0, rows0, gsem0)

    def _p2(t, _):
        b0 = 2 * t
        @pl.when(t > 0)
        def _drain1():
            _wait_scatter(rows1, ssem1)
        _start_gather(b0 + 1, rows1, gsem1)
        _wait_gather(rows0, gsem0)
        _scale_rows(rows0, b0)
        _start_scatter(b0, rows0, ssem0)
        _wait_scatter(rows0, ssem0)
        @pl.when(t < NB // 2 - 1)
        def _next():
            _start_gather(b0 + 2, rows0, gsem0)
        _wait_gather(rows1, gsem1)
        _scale_rows(rows1, b0 + 1)
        _start_scatter(b0 + 1, rows1, ssem1)
        return 0

    lax.fori_loop(0, NB // 2, _p2, 0)
    _wait_scatter(rows1, ssem1)

    plsc.subcore_barrier()

    @pl.when(s < 10)
    def _export_acc():
        pltpu.sync_copy(acc_sp.at[pl.ds(s * 1000, 1000)],
                        acc_hbm.at[c, pl.ds(s * 1000, 1000)])

    @pl.when(jnp.logical_and(s == 0, c == 0))
    def _export_den():
        pltpu.sync_copy(den_sp, den_hbm)


def _sc_edge(xh, src, dst, asrc, adst):
    f = pl.kernel(
        _sc_edge_body,
        out_type=[
            jax.ShapeDtypeStruct((NC, N, CH), jnp.float32),
            jax.ShapeDtypeStruct((N,), jnp.float32),
        ],
        mesh=plsc.VectorSubcoreMesh(core_axis_name="c", subcore_axis_name="s"),
        compiler_params=pltpu.CompilerParams(use_tc_tiling_on_sc=False),
        scratch_types=[
            pltpu.VMEM((NB, K), jnp.int32),
            pltpu.VMEM((NB, K), jnp.int32),
            pltpu.VMEM((2000,), jnp.float32),
            pltpu.VMEM((2000,), jnp.float32),
            pltpu.VMEM((EW,), jnp.float32),
            pltpu.VMEM((K, CH), jnp.float32),
            pltpu.VMEM((K, CH), jnp.float32),
            pltpu.VMEM((200, CH), jnp.float32),
            pltpu.VMEM((2000,), jnp.float32),
            pltpu.VMEM_SHARED((N, CH), jnp.float32),
            pltpu.VMEM_SHARED((N,), jnp.float32),
            pltpu.SemaphoreType.DMA,
            pltpu.SemaphoreType.DMA,
            pltpu.SemaphoreType.DMA,
            pltpu.SemaphoreType.DMA,
            pltpu.SemaphoreType.DMA,
        ],
    )
    return f(xh, src, dst, asrc, adst)


def kernel(x, edge_index, W1, att_src1, att_dst1, bias1,
           W2, att_src2, att_dst2, bias2):
    ei = edge_index.astype(jnp.int32).reshape(2, NS, NB, K)
    src, dst = ei[0], ei[1]

    xh1, asrc1, adst1 = _tc_prep(x, W1, att_src1, att_dst1)
    acc1, den1 = _sc_edge(xh1, src, dst, asrc1, adst1)
    xh2, asrc2, adst2 = _tc_mid(acc1, den1, bias1, W2, att_src2, att_dst2)
    acc2, den2 = _sc_edge(xh2, src, dst, asrc2, adst2)
    return _tc_final(acc2, den2, bias2)
